# baseline (speedup 1.0000x reference)
import functools

import jax
import jax.numpy as jnp
from jax import lax
from jax.experimental import pallas as pl
from jax.experimental.pallas import tpu as pltpu
from jax.experimental.pallas import tpu_sc as plsc

_NC = 2
_NS = 16
_NW = _NC * _NS
_LANES = 16


@functools.lru_cache(maxsize=None)
def _build_gather(n_tok: int, vocab: int, dim: int, chunk: int):
    assert n_tok % (_NW * chunk) == 0
    per_tile = n_tok // _NW
    n_chunks = per_tile // chunk
    row_out = 4 * dim

    mesh = plsc.VectorSubcoreMesh(core_axis_name="c", subcore_axis_name="s")

    @functools.partial(
        pl.kernel,
        mesh=mesh,
        out_type=jax.ShapeDtypeStruct((n_tok * row_out,), jnp.float32),
        compiler_params=pltpu.CompilerParams(needs_layout_passes=False,
                                             use_tc_tiling_on_sc=False),
        scratch_types=[
            pltpu.VMEM((chunk,), jnp.int32),
            pltpu.VMEM((4, chunk, dim), jnp.float32),
            pltpu.VMEM((chunk * row_out,), jnp.float32),
            pltpu.SemaphoreType.DMA,
        ],
    )
    def gather_kernel(x_hbm, t0, t1, t2, t3, out_hbm, idx_v, rall, o_v, sem):
        wid = lax.axis_index("s") * _NC + lax.axis_index("c")
        tile_base = wid * per_tile
        iota = lax.iota(jnp.int32, _LANES)
        t_lane = lax.rem(iota, 4)
        d_lanes = [iota // 4 + j * 4 for j in range(row_out // _LANES)]

        def chunk_body(ci, _):
            tok0 = tile_base + ci * chunk
            pltpu.sync_copy(x_hbm.at[pl.ds(tok0, chunk)], idx_v)
            cps = [pltpu.async_copy(t.at[idx_v], rall.at[j], sem)
                   for j, t in enumerate((t0, t1, t2, t3))]
            for cp in cps:
                cp.wait()

            def tok_body(i, _):
                i_lane = jnp.full((_LANES,), 0, jnp.int32) + i
                for j in range(row_out // _LANES):
                    data = plsc.load_gather(rall, [t_lane, i_lane, d_lanes[j]])
                    o_v[pl.ds(i * row_out + j * _LANES, _LANES)] = data
                return 0

            lax.fori_loop(0, chunk, tok_body, 0, unroll=2)
            pltpu.sync_copy(o_v, out_hbm.at[pl.ds(tok0 * row_out,
                                                  chunk * row_out)])
            return 0

        lax.fori_loop(0, n_chunks, chunk_body, 0)

    return gather_kernel


def kernel(x, scalar, vector_i, vector_j, vector_k):
    b, l = x.shape
    vocab, dim = scalar.shape
    n_tok = b * l
    x_flat = x.reshape(n_tok).astype(jnp.int32)
    call = _build_gather(n_tok, vocab, dim, 256)
    out = call(x_flat, scalar, vector_i, vector_j, vector_k)
    return out.reshape(b, l, dim, 4)


# speedup vs baseline: 2.7444x; 2.7444x over previous
import functools

import jax
import jax.numpy as jnp
from jax import lax
from jax.experimental import pallas as pl
from jax.experimental.pallas import tpu as pltpu
from jax.experimental.pallas import tpu_sc as plsc

_NC = 2
_NS = 16
_NW = _NC * _NS
_LANES = 16
_BB = 128


@functools.lru_cache(maxsize=None)
def _build_gather(batch: int, seq: int, vocab: int, dim: int):
    assert batch == _NW * _BB
    n_bchunk = _BB // _LANES

    mesh = plsc.VectorSubcoreMesh(core_axis_name="c", subcore_axis_name="s")

    @functools.partial(
        pl.kernel,
        mesh=mesh,
        out_type=jax.ShapeDtypeStruct((seq, dim, _NW, 4, _BB), jnp.float32),
        compiler_params=pltpu.CompilerParams(needs_layout_passes=False,
                                             use_tc_tiling_on_sc=False),
        scratch_types=[
            pltpu.VMEM((_BB * seq,), jnp.int32),
            pltpu.VMEM((_BB,), jnp.int32),
            pltpu.VMEM((4, _BB, dim), jnp.float32),
            pltpu.VMEM((dim, 4, _BB), jnp.float32),
            pltpu.SemaphoreType.DMA,
        ],
    )
    def gather_kernel(x_hbm, t0, t1, t2, t3, out_hbm, xt_v, idx_v, rall,
                      o_v, sem):
        wid = lax.axis_index("s") * _NC + lax.axis_index("c")
        iota = lax.iota(jnp.int32, _LANES)
        rows = [iota + bc * _LANES for bc in range(n_bchunk)]

        pltpu.sync_copy(x_hbm.at[pl.ds(wid * (_BB * seq), _BB * seq)], xt_v)

        def seq_body(l, _):
            l_vec = jnp.full((_LANES,), 0, jnp.int32) + l
            for bc in range(n_bchunk):
                idx_v[pl.ds(bc * _LANES, _LANES)] = plsc.load_gather(
                    xt_v, [rows[bc] * seq + l_vec])
            cps = [pltpu.async_copy(t.at[idx_v], rall.at[j], sem)
                   for j, t in enumerate((t0, t1, t2, t3))]
            for cp in cps:
                cp.wait()

            def dim_body(d, _):
                d_vec = jnp.full((_LANES,), 0, jnp.int32) + d
                for t in range(4):
                    t_vec = jnp.full((_LANES,), t, jnp.int32)
                    for bc in range(n_bchunk):
                        v = plsc.load_gather(rall, [t_vec, rows[bc], d_vec])
                        o_v[d, t, pl.ds(bc * _LANES, _LANES)] = v
                return 0

            lax.fori_loop(0, dim, dim_body, 0)
            pltpu.sync_copy(o_v, out_hbm.at[l, :, wid])
            return 0

        lax.fori_loop(0, seq, seq_body, 0)

    return gather_kernel


def kernel(x, scalar, vector_i, vector_j, vector_k):
    b, l = x.shape
    vocab, dim = scalar.shape
    x_flat = x.reshape(b * l).astype(jnp.int32)
    call = _build_gather(b, l, vocab, dim)
    o5 = call(x_flat, scalar, vector_i, vector_j, vector_k)
    return o5.transpose(2, 4, 0, 1, 3).reshape(b, l, dim, 4)


# speedup vs baseline: 2.8237x; 1.0289x over previous
import functools

import jax
import jax.numpy as jnp
from jax import lax
from jax.experimental import pallas as pl
from jax.experimental.pallas import tpu as pltpu
from jax.experimental.pallas import tpu_sc as plsc

_NC = 2
_NS = 16
_NW = _NC * _NS
_LANES = 16
_BB = 128


@functools.lru_cache(maxsize=None)
def _build_gather(batch: int, seq: int, vocab: int, dim: int):
    assert batch == _NW * _BB and seq % 2 == 0
    n_bchunk = _BB // _LANES

    mesh = plsc.VectorSubcoreMesh(core_axis_name="c", subcore_axis_name="s")

    @functools.partial(
        pl.kernel,
        mesh=mesh,
        out_type=jax.ShapeDtypeStruct((seq, dim, _NW, 4, _BB), jnp.float32),
        compiler_params=pltpu.CompilerParams(needs_layout_passes=False,
                                             use_tc_tiling_on_sc=False),
        scratch_types=[
            pltpu.VMEM((_BB * seq,), jnp.int32),
            pltpu.VMEM((2, _BB), jnp.int32),
            pltpu.VMEM((2, 4, _BB, dim), jnp.float32),
            pltpu.VMEM((dim, 4, _BB), jnp.float32),
            pltpu.SemaphoreType.DMA,
            pltpu.SemaphoreType.DMA,
        ],
    )
    def gather_kernel(x_hbm, t0, t1, t2, t3, out_hbm, xt_v, idx_v, rall,
                      o_v, sem0, sem1):
        wid = lax.axis_index("s") * _NC + lax.axis_index("c")
        iota = lax.iota(jnp.int32, _LANES)
        rows = [iota + bc * _LANES for bc in range(n_bchunk)]
        tabs = (t0, t1, t2, t3)
        sems = (sem0, sem1)

        pltpu.sync_copy(x_hbm.at[pl.ds(wid * (_BB * seq), _BB * seq)], xt_v)

        def start_gather(l, buf):
            l_vec = jnp.full((_LANES,), 0, jnp.int32) + l
            for bc in range(n_bchunk):
                idx_v[buf, pl.ds(bc * _LANES, _LANES)] = plsc.load_gather(
                    xt_v, [rows[bc] * seq + l_vec])
            for j, t in enumerate(tabs):
                pltpu.async_copy(t.at[idx_v.at[buf]], rall.at[buf, j],
                                 sems[buf])

        def wait_gather(buf):
            for j, t in enumerate(tabs):
                pltpu.make_async_copy(t.at[idx_v.at[buf]], rall.at[buf, j],
                                      sems[buf]).wait()

        def interleave_and_store(l, buf):
            def dim_body(d, _):
                d_vec = jnp.full((_LANES,), 0, jnp.int32) + d
                for t in range(4):
                    t_vec = jnp.full((_LANES,), t, jnp.int32)
                    for bc in range(n_bchunk):
                        v = plsc.load_gather(rall.at[buf],
                                             [t_vec, rows[bc], d_vec])
                        o_v[d, t, pl.ds(bc * _LANES, _LANES)] = v
                return 0

            lax.fori_loop(0, dim, dim_body, 0)
            pltpu.sync_copy(o_v, out_hbm.at[l, :, wid])

        start_gather(0, 0)

        def seq_body(i2, _):
            for phase in range(2):
                l = i2 * 2 + phase
                wait_gather(phase)

                @pl.when(l + 1 < seq)
                def _():
                    start_gather(l + 1, 1 - phase)

                interleave_and_store(l, phase)
            return 0

        lax.fori_loop(0, seq // 2, seq_body, 0)

    return gather_kernel


def kernel(x, scalar, vector_i, vector_j, vector_k):
    b, l = x.shape
    vocab, dim = scalar.shape
    x_flat = x.reshape(b * l).astype(jnp.int32)
    call = _build_gather(b, l, vocab, dim)
    o5 = call(x_flat, scalar, vector_i, vector_j, vector_k)
    return o5.transpose(2, 4, 0, 1, 3).reshape(b, l, dim, 4)


# speedup vs baseline: 2.8634x; 1.0141x over previous
import functools

import jax
import jax.numpy as jnp
from jax import lax
from jax.experimental import pallas as pl
from jax.experimental.pallas import tpu as pltpu
from jax.experimental.pallas import tpu_sc as plsc

_NC = 2
_NS = 16
_NW = _NC * _NS
_LANES = 16
_BB = 128


@functools.lru_cache(maxsize=None)
def _build_gather(batch: int, seq: int, vocab: int, dim: int):
    assert batch == _NW * _BB and seq % 2 == 0
    n_bchunk = _BB // _LANES

    mesh = plsc.VectorSubcoreMesh(core_axis_name="c", subcore_axis_name="s")

    @functools.partial(
        pl.kernel,
        mesh=mesh,
        out_type=jax.ShapeDtypeStruct((seq, dim, _NW, 4, _BB), jnp.float32),
        compiler_params=pltpu.CompilerParams(needs_layout_passes=False,
                                             use_tc_tiling_on_sc=False),
        scratch_types=[
            pltpu.VMEM((_BB * seq,), jnp.int32),
            pltpu.VMEM((2, _BB), jnp.int32),
            pltpu.VMEM((2, 4 * _BB, dim), jnp.float32),
            pltpu.VMEM((2, dim, 4, _BB), jnp.float32),
            pltpu.SemaphoreType.DMA,
            pltpu.SemaphoreType.DMA,
            pltpu.SemaphoreType.DMA,
            pltpu.SemaphoreType.DMA,
        ],
    )
    def gather_kernel(x_hbm, t0, t1, t2, t3, out_hbm, xt_v, idx_v, rall,
                      o_v, semg0, semg1, semo0, semo1):
        wid = lax.axis_index("s") * _NC + lax.axis_index("c")
        iota = lax.iota(jnp.int32, _LANES)
        tabs = (t0, t1, t2, t3)
        semg = (semg0, semg1)
        semo = (semo0, semo1)
        xrows = [(iota + bc * _LANES) * seq for bc in range(n_bchunk)]
        rrows = [[iota + (t * _BB + bc * _LANES) for bc in range(n_bchunk)]
                 for t in range(4)]
        zeros = jnp.full((_LANES,), 0, jnp.int32)

        pltpu.sync_copy(x_hbm.at[pl.ds(wid * (_BB * seq), _BB * seq)], xt_v)

        def start_gather(l, buf):
            l_vec = zeros + l
            for bc in range(n_bchunk):
                idx_v[buf, pl.ds(bc * _LANES, _LANES)] = plsc.load_gather(
                    xt_v, [xrows[bc] + l_vec])
            for j, t in enumerate(tabs):
                pltpu.async_copy(
                    t.at[idx_v.at[buf]],
                    rall.at[buf, pl.ds(j * _BB, _BB), :], semg[buf])

        def wait_gather(buf):
            for j, t in enumerate(tabs):
                pltpu.make_async_copy(
                    t.at[idx_v.at[buf]],
                    rall.at[buf, pl.ds(j * _BB, _BB), :], semg[buf]).wait()

        def interleave(buf):
            rbuf = rall.at[buf]

            def dim_body(d, _):
                d_vec = zeros + d
                for t in range(4):
                    for bc in range(n_bchunk):
                        v = plsc.load_gather(rbuf, [rrows[t][bc], d_vec])
                        o_v[buf, d, t, pl.ds(bc * _LANES, _LANES)] = v
                return 0

            lax.fori_loop(0, dim, dim_body, 0, unroll=2)

        def start_out(l, buf):
            pltpu.async_copy(o_v.at[buf], out_hbm.at[l, :, wid], semo[buf])

        def wait_out(l, buf):
            pltpu.make_async_copy(o_v.at[buf], out_hbm.at[l, :, wid],
                                  semo[buf]).wait()

        start_gather(0, 0)

        def seq_body(i2, _):
            for phase in range(2):
                l = i2 * 2 + phase
                wait_gather(phase)

                @pl.when(l + 1 < seq)
                def _():
                    start_gather(l + 1, 1 - phase)

                @pl.when(i2 > 0)
                def _():
                    wait_out(l - 2, phase)

                interleave(phase)
                start_out(l, phase)
            return 0

        lax.fori_loop(0, seq // 2, seq_body, 0)
        wait_out(seq - 2, 0)
        wait_out(seq - 1, 1)

    return gather_kernel


def kernel(x, scalar, vector_i, vector_j, vector_k):
    b, l = x.shape
    vocab, dim = scalar.shape
    x_flat = x.reshape(b * l).astype(jnp.int32)
    call = _build_gather(b, l, vocab, dim)
    o5 = call(x_flat, scalar, vector_i, vector_j, vector_k)
    return o5.transpose(2, 4, 0, 1, 3).reshape(b, l, dim, 4)


# speedup vs baseline: 3.2818x; 1.1461x over previous
import functools

import jax
import jax.numpy as jnp
from jax import lax
from jax.experimental import pallas as pl
from jax.experimental.pallas import tpu as pltpu
from jax.experimental.pallas import tpu_sc as plsc

_NC = 2
_NS = 16
_NW = _NC * _NS
_LANES = 16
_BB = 128


@functools.lru_cache(maxsize=None)
def _build_gather(batch: int, seq: int, vocab: int, dim: int):
    assert batch == _NW * _BB and seq % 2 == 0
    n_bchunk = _BB // _LANES

    mesh = plsc.VectorSubcoreMesh(core_axis_name="c", subcore_axis_name="s")

    @functools.partial(
        pl.kernel,
        mesh=mesh,
        out_type=jax.ShapeDtypeStruct((seq, dim, _NW, 4, _BB), jnp.float32),
        compiler_params=pltpu.CompilerParams(needs_layout_passes=False,
                                             use_tc_tiling_on_sc=False),
        scratch_types=[
            pltpu.VMEM((_BB * seq,), jnp.int32),
            pltpu.VMEM((2, _BB), jnp.int32),
            pltpu.VMEM((2, 4 * _BB, dim), jnp.float32),
            pltpu.VMEM((2, dim, 4, _BB), jnp.float32),
            pltpu.SemaphoreType.DMA,
            pltpu.SemaphoreType.DMA,
            pltpu.SemaphoreType.DMA,
            pltpu.SemaphoreType.DMA,
        ],
    )
    def gather_kernel(x_hbm, t0, t1, t2, t3, out_hbm, xt_v, idx_v, rall,
                      o_v, semg0, semg1, semo0, semo1):
        wid = lax.axis_index("s") * _NC + lax.axis_index("c")
        iota = lax.iota(jnp.int32, _LANES)
        tabs = (t0, t1, t2, t3)
        semg = (semg0, semg1)
        semo = (semo0, semo1)
        xrows = [(iota + bc * _LANES) * seq for bc in range(n_bchunk)]
        rrows = [[iota + (t * _BB + bc * _LANES) for bc in range(n_bchunk)]
                 for t in range(4)]
        zeros = jnp.full((_LANES,), 0, jnp.int32)

        pltpu.sync_copy(x_hbm.at[pl.ds(wid * (_BB * seq), _BB * seq)], xt_v)

        def start_gather(l, buf):
            l_vec = zeros + l
            for bc in range(n_bchunk):
                idx_v[buf, pl.ds(bc * _LANES, _LANES)] = plsc.load_gather(
                    xt_v, [xrows[bc] + l_vec])
            for j, t in enumerate(tabs):
                pltpu.async_copy(
                    t.at[idx_v.at[buf]],
                    rall.at[buf, pl.ds(j * _BB, _BB), :], semg[buf])

        def wait_gather(buf):
            for j, t in enumerate(tabs):
                pltpu.make_async_copy(
                    t.at[idx_v.at[buf]],
                    rall.at[buf, pl.ds(j * _BB, _BB), :], semg[buf]).wait()

        def interleave(buf):
            rbuf = rall.at[buf]

            @plsc.parallel_loop(0, dim, unroll=4)
            def dim_body(d):
                d_vec = zeros + d
                for t in range(4):
                    for bc in range(n_bchunk):
                        v = plsc.load_gather(rbuf, [rrows[t][bc], d_vec])
                        o_v[buf, d, t, pl.ds(bc * _LANES, _LANES)] = v

        def start_out(l, buf):
            pltpu.async_copy(o_v.at[buf], out_hbm.at[l, :, wid], semo[buf])

        def wait_out(l, buf):
            pltpu.make_async_copy(o_v.at[buf], out_hbm.at[l, :, wid],
                                  semo[buf]).wait()

        start_gather(0, 0)

        def seq_body(i2, _):
            for phase in range(2):
                l = i2 * 2 + phase
                wait_gather(phase)

                @pl.when(l + 1 < seq)
                def _():
                    start_gather(l + 1, 1 - phase)

                @pl.when(i2 > 0)
                def _():
                    wait_out(l - 2, phase)

                interleave(phase)
                start_out(l, phase)
            return 0

        lax.fori_loop(0, seq // 2, seq_body, 0)
        wait_out(seq - 2, 0)
        wait_out(seq - 1, 1)

    return gather_kernel


def kernel(x, scalar, vector_i, vector_j, vector_k):
    b, l = x.shape
    vocab, dim = scalar.shape
    x_flat = x.reshape(b * l).astype(jnp.int32)
    call = _build_gather(b, l, vocab, dim)
    o5 = call(x_flat, scalar, vector_i, vector_j, vector_k)
    return o5.transpose(2, 4, 0, 1, 3).reshape(b, l, dim, 4)
